# baseline (speedup 1.0000x reference)
import functools

import jax
import jax.numpy as jnp
from jax import lax
from jax.experimental import pallas as pl
from jax.experimental.pallas import tpu as pltpu
from jax.experimental.pallas import tpu_sc as plsc

VOCAB = 100000
EMB = 64
EPS = 1e-5

NC = 2
NS = 16
NW = NC * NS

ROW_BLOCK = 5000

BCH = 128
NBUF = 4


def _ln_table_body(w_ref, g_ref, b_ref, o_ref):
    x = w_ref[...]
    mean = jnp.mean(x, axis=-1, keepdims=True)
    xc = x - mean
    var = jnp.mean(xc * xc, axis=-1, keepdims=True)
    n = xc * lax.rsqrt(var + EPS) * g_ref[...] + b_ref[...]
    o_ref[...] = jnp.concatenate([n, jnp.zeros_like(n)], axis=-1)


def _normalize_table(table, gamma, beta):
    grid = VOCAB // ROW_BLOCK
    return pl.pallas_call(
        _ln_table_body,
        grid=(grid,),
        in_specs=[
            pl.BlockSpec((ROW_BLOCK, EMB), lambda i: (i, 0)),
            pl.BlockSpec((1, EMB), lambda i: (0, 0)),
            pl.BlockSpec((1, EMB), lambda i: (0, 0)),
        ],
        out_specs=pl.BlockSpec((ROW_BLOCK, 2 * EMB), lambda i: (i, 0)),
        out_shape=jax.ShapeDtypeStruct((VOCAB, 2 * EMB), jnp.float32),
    )(table, gamma.reshape(1, EMB), beta.reshape(1, EMB))


def _make_gather(batch, seq):
    mesh = plsc.VectorSubcoreMesh(core_axis_name="c", subcore_axis_name="s")

    @functools.partial(
        pl.kernel,
        mesh=mesh,
        compiler_params=pltpu.CompilerParams(needs_layout_passes=False),
        out_type=jax.ShapeDtypeStruct((seq, EMB, batch), jnp.float32),
        scratch_types=(
            [pltpu.VMEM((seq, BCH), jnp.int32)]
            + [pltpu.VMEM((BCH, 2 * EMB), jnp.float32)] * NBUF
            + [pltpu.SemaphoreType.DMA] * (2 * NBUF)
        ),
    )
    def gather_kernel(table_hbm, idxt_hbm, out_hbm, idx_tv, *bufs_sems):
        rows = bufs_sems[:NBUF]
        gsems = bufs_sems[NBUF:2 * NBUF]
        wsems = bufs_sems[2 * NBUF:]
        wid = lax.axis_index("s") * NC + lax.axis_index("c")
        b0 = wid * BCH
        pltpu.sync_copy(idxt_hbm.at[:, pl.ds(b0, BCH)], idx_tv)

        pltpu.async_copy(table_hbm.at[idx_tv.at[0]], rows[0], gsems[0])
        pltpu.async_copy(table_hbm.at[idx_tv.at[1]], rows[1], gsems[1])

        def body(i, _):
            j0 = i * NBUF
            for t in range(NBUF):
                j = j0 + t
                pltpu.make_async_copy(
                    table_hbm.at[pl.ds(0, BCH)], rows[t], gsems[t]).wait()

                tp = (t + NBUF - 1) % NBUF
                @pl.when(j >= 1)
                def _():
                    pltpu.make_async_copy(
                        rows[tp].at[pl.ds(0, EMB)],
                        out_hbm.at[0, :, pl.ds(0, BCH)], wsems[tp]).wait()

                for e in range(EMB):
                    pltpu.async_copy(
                        rows[t].at[:, e], out_hbm.at[j, e, pl.ds(b0, BCH)],
                        wsems[t])

                t2 = (t + 2) % NBUF
                @pl.when(j + 2 < seq)
                def _():
                    pltpu.async_copy(
                        table_hbm.at[idx_tv.at[j + 2]], rows[t2], gsems[t2])
            return 0

        lax.fori_loop(0, seq // NBUF, body, 0)
        tl = (seq - 1) % NBUF
        pltpu.make_async_copy(
            rows[tl].at[pl.ds(0, EMB)],
            out_hbm.at[0, :, pl.ds(0, BCH)], wsems[tl]).wait()

    return gather_kernel


def kernel(input_tensor, token_emb_weight, ln_weight, ln_bias):
    batch, seq = input_tensor.shape
    normed = _normalize_table(token_emb_weight, ln_weight, ln_bias)
    idx_t = jnp.transpose(input_tensor)
    out_t = _make_gather(batch, seq)(normed, idx_t)
    return jnp.transpose(out_t, (2, 0, 1))


# speedup vs baseline: 335.3753x; 335.3753x over previous
import functools

import jax
import jax.numpy as jnp
from jax import lax
from jax.experimental import pallas as pl
from jax.experimental.pallas import tpu as pltpu
from jax.experimental.pallas import tpu_sc as plsc

VOCAB = 100000
EMB = 64
EPS = 1e-5

NC = 2
NS = 16
NW = NC * NS

ROW_BLOCK = 5000

BCH = 128


def _ln_table_body(w_ref, g_ref, b_ref, o_ref):
    x = w_ref[...]
    mean = jnp.mean(x, axis=-1, keepdims=True)
    xc = x - mean
    var = jnp.mean(xc * xc, axis=-1, keepdims=True)
    n = xc * lax.rsqrt(var + EPS) * g_ref[...] + b_ref[...]
    o_ref[...] = jnp.concatenate([n, jnp.zeros_like(n)], axis=-1)


def _normalize_table(table, gamma, beta):
    grid = VOCAB // ROW_BLOCK
    return pl.pallas_call(
        _ln_table_body,
        grid=(grid,),
        in_specs=[
            pl.BlockSpec((ROW_BLOCK, EMB), lambda i: (i, 0)),
            pl.BlockSpec((1, EMB), lambda i: (0, 0)),
            pl.BlockSpec((1, EMB), lambda i: (0, 0)),
        ],
        out_specs=pl.BlockSpec((ROW_BLOCK, 2 * EMB), lambda i: (i, 0)),
        out_shape=jax.ShapeDtypeStruct((VOCAB, 2 * EMB), jnp.float32),
    )(table, gamma.reshape(1, EMB), beta.reshape(1, EMB))


def _make_gather(batch, seq):
    mesh = plsc.VectorSubcoreMesh(core_axis_name="c", subcore_axis_name="s")

    @functools.partial(
        pl.kernel,
        mesh=mesh,
        compiler_params=pltpu.CompilerParams(needs_layout_passes=False),
        out_type=jax.ShapeDtypeStruct((seq, EMB, batch), jnp.float32),
        scratch_types=[
            pltpu.VMEM((seq, BCH), jnp.int32),
            pltpu.VMEM((2, BCH, 2 * EMB), jnp.float32),
            pltpu.VMEM((2, EMB, BCH), jnp.float32),
            pltpu.SemaphoreType.DMA,
            pltpu.SemaphoreType.DMA,
            pltpu.SemaphoreType.DMA,
            pltpu.SemaphoreType.DMA,
        ],
    )
    def gather_kernel(table_hbm, idxt_hbm, out_hbm, idx_tv, rows_v,
                      trans_v, gsem0, gsem1, wsem0, wsem1):
        gsems = (gsem0, gsem1)
        wsems = (wsem0, wsem1)
        wid = lax.axis_index("s") * NC + lax.axis_index("c")
        b0 = wid * BCH
        pltpu.sync_copy(idxt_hbm.at[:, pl.ds(b0, BCH)], idx_tv)

        lanes = lax.broadcasted_iota(jnp.int32, (16,), 0)
        rowvs = [lanes + g * 16 for g in range(BCH // 16)]

        pltpu.async_copy(table_hbm.at[idx_tv.at[0]], rows_v.at[0], gsem0)
        pltpu.async_copy(table_hbm.at[idx_tv.at[1]], rows_v.at[1], gsem1)

        def transpose(b):
            def dbody(d, _):
                rot = (d + lanes) & 15
                cols = [rot + e_blk * 16 for e_blk in range(EMB // 16)]
                for e_blk in range(EMB // 16):
                    col = cols[e_blk]
                    vs = [plsc.load_gather(rows_v.at[b], [rowvs[g], col])
                          for g in range(BCH // 16)]
                    for g in range(BCH // 16):
                        plsc.store_scatter(trans_v.at[b], [col, rowvs[g]],
                                           vs[g])
                return 0
            lax.fori_loop(0, 16, dbody, 0)

        def body(i, _):
            s0 = i * 2
            for b in range(2):
                s = s0 + b
                pltpu.make_async_copy(
                    table_hbm.at[pl.ds(0, BCH)], rows_v.at[b], gsems[b]).wait()

                @pl.when(s >= 2)
                def _():
                    pltpu.make_async_copy(
                        trans_v.at[b],
                        out_hbm.at[0, :, pl.ds(0, BCH)], wsems[b]).wait()

                transpose(b)
                pltpu.async_copy(
                    trans_v.at[b], out_hbm.at[s, :, pl.ds(b0, BCH)], wsems[b])

                @pl.when(s + 2 < seq)
                def _():
                    pltpu.async_copy(
                        table_hbm.at[idx_tv.at[s + 2]], rows_v.at[b], gsems[b])
            return 0

        lax.fori_loop(0, seq // 2, body, 0)
        for b in range(2):
            pltpu.make_async_copy(
                trans_v.at[b], out_hbm.at[0, :, pl.ds(0, BCH)], wsems[b]).wait()

    return gather_kernel


def kernel(input_tensor, token_emb_weight, ln_weight, ln_bias):
    batch, seq = input_tensor.shape
    normed = _normalize_table(token_emb_weight, ln_weight, ln_bias)
    idx_t = jnp.transpose(input_tensor)
    out_t = _make_gather(batch, seq)(normed, idx_t)
    return jnp.transpose(out_t, (2, 0, 1))


# speedup vs baseline: 344.4852x; 1.0272x over previous
import functools

import jax
import jax.numpy as jnp
from jax import lax
from jax.experimental import pallas as pl
from jax.experimental.pallas import tpu as pltpu
from jax.experimental.pallas import tpu_sc as plsc

VOCAB = 100000
EMB = 64
EPS = 1e-5

NC = 2
NS = 16
NW = NC * NS

VPAD = 102400
ROW_BLOCK = 2048

BCH = 128


def _ln_table_body(w_ref, g_ref, b_ref, o_ref):
    x = w_ref[...]
    mean = jnp.mean(x, axis=0, keepdims=True)
    xc = x - mean
    var = jnp.mean(xc * xc, axis=0, keepdims=True)
    n = xc * lax.rsqrt(var + EPS) * g_ref[...] + b_ref[...]
    t = jnp.transpose(n)
    o_ref[...] = jnp.concatenate([t, jnp.zeros_like(t)], axis=-1)


def _normalize_table(table_t, gamma, beta):
    grid = VPAD // ROW_BLOCK
    return pl.pallas_call(
        _ln_table_body,
        grid=(grid,),
        in_specs=[
            pl.BlockSpec((EMB, ROW_BLOCK), lambda i: (0, i)),
            pl.BlockSpec((EMB, 1), lambda i: (0, 0)),
            pl.BlockSpec((EMB, 1), lambda i: (0, 0)),
        ],
        out_specs=pl.BlockSpec((ROW_BLOCK, 2 * EMB), lambda i: (i, 0)),
        out_shape=jax.ShapeDtypeStruct((VPAD, 2 * EMB), jnp.float32),
    )(table_t, gamma.reshape(EMB, 1), beta.reshape(EMB, 1))


def _make_gather(batch, seq):
    mesh = plsc.VectorSubcoreMesh(core_axis_name="c", subcore_axis_name="s")

    @functools.partial(
        pl.kernel,
        mesh=mesh,
        compiler_params=pltpu.CompilerParams(needs_layout_passes=False),
        out_type=jax.ShapeDtypeStruct((seq, EMB, batch), jnp.float32),
        scratch_types=[
            pltpu.VMEM((seq, BCH), jnp.int32),
            pltpu.VMEM((2, BCH, 2 * EMB), jnp.float32),
            pltpu.VMEM((2, EMB, BCH), jnp.float32),
            pltpu.SemaphoreType.DMA,
            pltpu.SemaphoreType.DMA,
            pltpu.SemaphoreType.DMA,
            pltpu.SemaphoreType.DMA,
        ],
    )
    def gather_kernel(table_hbm, idxt_hbm, out_hbm, idx_tv, rows_v,
                      trans_v, gsem0, gsem1, wsem0, wsem1):
        gsems = (gsem0, gsem1)
        wsems = (wsem0, wsem1)
        wid = lax.axis_index("s") * NC + lax.axis_index("c")
        b0 = wid * BCH
        pltpu.sync_copy(idxt_hbm.at[:, pl.ds(b0, BCH)], idx_tv)

        lanes = lax.broadcasted_iota(jnp.int32, (16,), 0)
        rowvs = [lanes + g * 16 for g in range(BCH // 16)]

        pltpu.async_copy(table_hbm.at[idx_tv.at[0]], rows_v.at[0], gsem0)
        pltpu.async_copy(table_hbm.at[idx_tv.at[1]], rows_v.at[1], gsem1)

        def transpose(b):
            def dbody(d, _):
                rot = (d + lanes) & 15
                cols = [rot + e_blk * 16 for e_blk in range(EMB // 16)]
                for e_blk in range(EMB // 16):
                    col = cols[e_blk]
                    vs = [plsc.load_gather(rows_v.at[b], [rowvs[g], col])
                          for g in range(BCH // 16)]
                    for g in range(BCH // 16):
                        plsc.store_scatter(trans_v.at[b], [col, rowvs[g]],
                                           vs[g])
                return 0
            lax.fori_loop(0, 16, dbody, 0)

        def body(i, _):
            s0 = i * 2
            for b in range(2):
                s = s0 + b
                pltpu.make_async_copy(
                    table_hbm.at[pl.ds(0, BCH)], rows_v.at[b], gsems[b]).wait()

                @pl.when(s >= 2)
                def _():
                    pltpu.make_async_copy(
                        trans_v.at[b],
                        out_hbm.at[0, :, pl.ds(0, BCH)], wsems[b]).wait()

                transpose(b)
                pltpu.async_copy(
                    trans_v.at[b], out_hbm.at[s, :, pl.ds(b0, BCH)], wsems[b])

                @pl.when(s + 2 < seq)
                def _():
                    pltpu.async_copy(
                        table_hbm.at[idx_tv.at[s + 2]], rows_v.at[b], gsems[b])
            return 0

        lax.fori_loop(0, seq // 2, body, 0)
        for b in range(2):
            pltpu.make_async_copy(
                trans_v.at[b], out_hbm.at[0, :, pl.ds(0, BCH)], wsems[b]).wait()

    return gather_kernel


def kernel(input_tensor, token_emb_weight, ln_weight, ln_bias):
    batch, seq = input_tensor.shape
    table_t = jnp.transpose(token_emb_weight)
    table_t = jnp.pad(table_t, ((0, 0), (0, VPAD - VOCAB)))
    normed = _normalize_table(table_t, ln_weight, ln_bias)
    idx_t = jnp.transpose(input_tensor)
    out_t = _make_gather(batch, seq)(normed, idx_t)
    return jnp.transpose(out_t, (2, 0, 1))


# speedup vs baseline: 389.5926x; 1.1309x over previous
import functools

import jax
import jax.numpy as jnp
from jax import lax
from jax.experimental import pallas as pl
from jax.experimental.pallas import tpu as pltpu
from jax.experimental.pallas import tpu_sc as plsc

VOCAB = 100000
EMB = 64
EPS = 1e-5

NC = 2
NS = 16
NW = NC * NS

VPAD = 102400
ROW_BLOCK = 2048

BCH = 128


def _ln_table_body(w_ref, g_ref, b_ref, o_ref):
    x = w_ref[...]
    mean = jnp.mean(x, axis=0, keepdims=True)
    xc = x - mean
    var = jnp.mean(xc * xc, axis=0, keepdims=True)
    n = xc * lax.rsqrt(var + EPS) * g_ref[...] + b_ref[...]
    t = jnp.transpose(n)
    o_ref[...] = jnp.concatenate([t, jnp.zeros_like(t)], axis=-1)


def _normalize_table(table_t, gamma, beta):
    grid = VPAD // ROW_BLOCK
    return pl.pallas_call(
        _ln_table_body,
        grid=(grid,),
        in_specs=[
            pl.BlockSpec((EMB, ROW_BLOCK), lambda i: (0, i)),
            pl.BlockSpec((EMB, 1), lambda i: (0, 0)),
            pl.BlockSpec((EMB, 1), lambda i: (0, 0)),
        ],
        out_specs=pl.BlockSpec((ROW_BLOCK, 2 * EMB), lambda i: (i, 0)),
        out_shape=jax.ShapeDtypeStruct((VPAD, 2 * EMB), jnp.float32),
    )(table_t, gamma.reshape(EMB, 1), beta.reshape(EMB, 1))


def _make_gather(batch, seq):
    mesh = plsc.VectorSubcoreMesh(core_axis_name="c", subcore_axis_name="s")

    @functools.partial(
        pl.kernel,
        mesh=mesh,
        compiler_params=pltpu.CompilerParams(needs_layout_passes=False),
        out_type=jax.ShapeDtypeStruct((seq, EMB, batch), jnp.float32),
        scratch_types=(
            [pltpu.VMEM((seq, BCH), jnp.int32)]
            + [pltpu.VMEM((BCH, 2 * EMB), jnp.float32)] * 4
            + [pltpu.VMEM((EMB, BCH), jnp.float32)] * 2
            + [pltpu.SemaphoreType.DMA] * 6
        ),
    )
    def gather_kernel(table_hbm, idxt_hbm, out_hbm, idx_tv, *rest):
        rows = rest[:4]
        trans = rest[4:6]
        gsems = rest[6:10]
        wsems = rest[10:12]
        wid = lax.axis_index("s") * NC + lax.axis_index("c")
        b0 = wid * BCH
        pltpu.sync_copy(idxt_hbm.at[:, pl.ds(b0, BCH)], idx_tv)

        lanes = lax.broadcasted_iota(jnp.int32, (16,), 0)
        rowvs = [lanes + g * 16 for g in range(BCH // 16)]

        pltpu.async_copy(table_hbm.at[idx_tv.at[0]], rows[0], gsems[0])
        pltpu.async_copy(table_hbm.at[idx_tv.at[1]], rows[1], gsems[1])

        def transpose(src, dst):
            def dbody(d, _):
                rot = (d + lanes) & 15
                for e_blk in range(EMB // 16):
                    col = rot + e_blk * 16
                    vs = [plsc.load_gather(src, [rowvs[g], col])
                          for g in range(BCH // 16)]
                    for g in range(BCH // 16):
                        plsc.store_scatter(dst, [col, rowvs[g]], vs[g])
                return 0
            lax.fori_loop(0, 16, dbody, 0)

        def body(i, _):
            j0 = i * 4
            for t in range(4):
                j = j0 + t
                tb = t % 2
                pltpu.make_async_copy(
                    table_hbm.at[pl.ds(0, BCH)], rows[t], gsems[t]).wait()

                t2 = (t + 2) % 4
                @pl.when(j + 2 < seq)
                def _():
                    pltpu.async_copy(
                        table_hbm.at[idx_tv.at[j + 2]], rows[t2], gsems[t2])

                @pl.when(j >= 2)
                def _():
                    pltpu.make_async_copy(
                        trans[tb],
                        out_hbm.at[0, :, pl.ds(0, BCH)], wsems[tb]).wait()

                transpose(rows[t], trans[tb])
                pltpu.async_copy(
                    trans[tb], out_hbm.at[j, :, pl.ds(b0, BCH)], wsems[tb])
            return 0

        lax.fori_loop(0, seq // 4, body, 0)
        for tb in range(2):
            pltpu.make_async_copy(
                trans[tb], out_hbm.at[0, :, pl.ds(0, BCH)], wsems[tb]).wait()

    return gather_kernel


def kernel(input_tensor, token_emb_weight, ln_weight, ln_bias):
    batch, seq = input_tensor.shape
    table_t = jnp.transpose(token_emb_weight)
    table_t = jnp.pad(table_t, ((0, 0), (0, VPAD - VOCAB)))
    normed = _normalize_table(table_t, ln_weight, ln_bias)
    idx_t = jnp.transpose(input_tensor)
    out_t = _make_gather(batch, seq)(normed, idx_t)
    return jnp.transpose(out_t, (2, 0, 1))


# speedup vs baseline: 433.0002x; 1.1114x over previous
import functools

import jax
import jax.numpy as jnp
from jax import lax
from jax.experimental import pallas as pl
from jax.experimental.pallas import tpu as pltpu
from jax.experimental.pallas import tpu_sc as plsc

VOCAB = 100000
EMB = 64
EPS = 1e-5

NC = 2
NS = 16
NW = NC * NS

ROW_BLOCK = 4096

BCH = 128


def _ln_table_body(w_ref, g_ref, b_ref, o_ref):
    x = w_ref[...]
    mean = jnp.mean(x, axis=0, keepdims=True)
    xc = x - mean
    var = jnp.mean(xc * xc, axis=0, keepdims=True)
    n = xc * lax.rsqrt(var + EPS) * g_ref[...] + b_ref[...]
    t = jnp.transpose(n)
    o_ref[...] = jnp.concatenate([t, jnp.zeros_like(t)], axis=-1)


def _normalize_table(table_t, gamma, beta):
    grid = (VOCAB + ROW_BLOCK - 1) // ROW_BLOCK
    return pl.pallas_call(
        _ln_table_body,
        grid=(grid,),
        in_specs=[
            pl.BlockSpec((EMB, ROW_BLOCK), lambda i: (0, i)),
            pl.BlockSpec((EMB, 1), lambda i: (0, 0)),
            pl.BlockSpec((EMB, 1), lambda i: (0, 0)),
        ],
        out_specs=pl.BlockSpec((ROW_BLOCK, 2 * EMB), lambda i: (i, 0)),
        out_shape=jax.ShapeDtypeStruct((VOCAB, 2 * EMB), jnp.float32),
    )(table_t, gamma.reshape(EMB, 1), beta.reshape(EMB, 1))


def _make_gather(batch, seq):
    mesh = plsc.VectorSubcoreMesh(core_axis_name="c", subcore_axis_name="s")

    @functools.partial(
        pl.kernel,
        mesh=mesh,
        compiler_params=pltpu.CompilerParams(needs_layout_passes=False),
        out_type=jax.ShapeDtypeStruct((seq, EMB, batch), jnp.float32),
        scratch_types=(
            [pltpu.VMEM((seq, BCH), jnp.int32)]
            + [pltpu.VMEM((BCH, 2 * EMB), jnp.float32)] * 4
            + [pltpu.VMEM((EMB, BCH), jnp.float32)] * 2
            + [pltpu.SemaphoreType.DMA] * 6
        ),
    )
    def gather_kernel(table_hbm, idxt_hbm, out_hbm, idx_tv, *rest):
        rows = rest[:4]
        trans = rest[4:6]
        gsems = rest[6:10]
        wsems = rest[10:12]
        wid = lax.axis_index("s") * NC + lax.axis_index("c")
        b0 = wid * BCH
        pltpu.sync_copy(idxt_hbm.at[:, pl.ds(b0, BCH)], idx_tv)

        lanes = lax.broadcasted_iota(jnp.int32, (16,), 0)
        rowvs = [lanes + g * 16 for g in range(BCH // 16)]

        pltpu.async_copy(table_hbm.at[idx_tv.at[0]], rows[0], gsems[0])
        pltpu.async_copy(table_hbm.at[idx_tv.at[1]], rows[1], gsems[1])

        def transpose(src, dst):
            def dbody(d, _):
                rot = (d + lanes) & 15
                for e_blk in range(EMB // 16):
                    col = rot + e_blk * 16
                    vs = [plsc.load_gather(src, [rowvs[g], col])
                          for g in range(BCH // 16)]
                    for g in range(BCH // 16):
                        plsc.store_scatter(dst, [col, rowvs[g]], vs[g])
                return 0
            lax.fori_loop(0, 16, dbody, 0)

        def body(i, _):
            j0 = i * 4
            for t in range(4):
                j = j0 + t
                tb = t % 2
                pltpu.make_async_copy(
                    table_hbm.at[pl.ds(0, BCH)], rows[t], gsems[t]).wait()

                t2 = (t + 2) % 4
                @pl.when(j + 2 < seq)
                def _():
                    pltpu.async_copy(
                        table_hbm.at[idx_tv.at[j + 2]], rows[t2], gsems[t2])

                @pl.when(j >= 2)
                def _():
                    pltpu.make_async_copy(
                        trans[tb],
                        out_hbm.at[0, :, pl.ds(0, BCH)], wsems[tb]).wait()

                transpose(rows[t], trans[tb])
                pltpu.async_copy(
                    trans[tb], out_hbm.at[j, :, pl.ds(b0, BCH)], wsems[tb])
            return 0

        lax.fori_loop(0, seq // 4, body, 0)
        for tb in range(2):
            pltpu.make_async_copy(
                trans[tb], out_hbm.at[0, :, pl.ds(0, BCH)], wsems[tb]).wait()

    return gather_kernel


def kernel(input_tensor, token_emb_weight, ln_weight, ln_bias):
    batch, seq = input_tensor.shape
    table_t = jnp.transpose(token_emb_weight)
    normed = _normalize_table(table_t, ln_weight, ln_bias)
    idx_t = jnp.transpose(input_tensor)
    out_t = _make_gather(batch, seq)(normed, idx_t)
    return jnp.transpose(out_t, (2, 0, 1))


# speedup vs baseline: 453.3174x; 1.0469x over previous
import functools

import jax
import jax.numpy as jnp
from jax import lax
from jax.experimental import pallas as pl
from jax.experimental.pallas import tpu as pltpu
from jax.experimental.pallas import tpu_sc as plsc

VOCAB = 100000
EMB = 64
EPS = 1e-5

NC = 2
NS = 16
NW = NC * NS

ROW_BLOCK = 8192

BCH = 128


def _ln_table_body(w_ref, g_ref, b_ref, o_ref):
    x = w_ref[...]
    mean = jnp.mean(x, axis=0, keepdims=True)
    xc = x - mean
    var = jnp.mean(xc * xc, axis=0, keepdims=True)
    n = xc * lax.rsqrt(var + EPS) * g_ref[...] + b_ref[...]
    t = jnp.transpose(n)
    o_ref[...] = jnp.concatenate([t, jnp.zeros_like(t)], axis=-1)


def _normalize_table(table_t, gamma, beta):
    grid = (VOCAB + ROW_BLOCK - 1) // ROW_BLOCK
    return pl.pallas_call(
        _ln_table_body,
        grid=(grid,),
        in_specs=[
            pl.BlockSpec((EMB, ROW_BLOCK), lambda i: (0, i)),
            pl.BlockSpec((EMB, 1), lambda i: (0, 0)),
            pl.BlockSpec((EMB, 1), lambda i: (0, 0)),
        ],
        out_specs=pl.BlockSpec((ROW_BLOCK, 2 * EMB), lambda i: (i, 0)),
        out_shape=jax.ShapeDtypeStruct((VOCAB, 2 * EMB), jnp.float32),
    )(table_t, gamma.reshape(EMB, 1), beta.reshape(EMB, 1))


def _make_gather(batch, seq):
    mesh = plsc.VectorSubcoreMesh(core_axis_name="c", subcore_axis_name="s")

    @functools.partial(
        pl.kernel,
        mesh=mesh,
        compiler_params=pltpu.CompilerParams(needs_layout_passes=False),
        out_type=jax.ShapeDtypeStruct((seq, EMB, batch), jnp.float32),
        scratch_types=(
            [pltpu.VMEM((seq, BCH), jnp.int32)]
            + [pltpu.VMEM((BCH, 2 * EMB), jnp.float32)] * 4
            + [pltpu.VMEM((EMB, BCH), jnp.float32)] * 2
            + [pltpu.SemaphoreType.DMA] * 6
        ),
    )
    def gather_kernel(table_hbm, idxt_hbm, out_hbm, idx_tv, *rest):
        rows = rest[:4]
        trans = rest[4:6]
        gsems = rest[6:10]
        wsems = rest[10:12]
        wid = lax.axis_index("s") * NC + lax.axis_index("c")
        b0 = wid * BCH
        pltpu.sync_copy(idxt_hbm.at[:, pl.ds(b0, BCH)], idx_tv)

        lanes = lax.broadcasted_iota(jnp.int32, (16,), 0)
        rowvs = [lanes + g * 16 for g in range(BCH // 16)]

        pltpu.async_copy(table_hbm.at[idx_tv.at[0]], rows[0], gsems[0])
        pltpu.async_copy(table_hbm.at[idx_tv.at[1]], rows[1], gsems[1])
        pltpu.async_copy(table_hbm.at[idx_tv.at[2]], rows[2], gsems[2])

        def transpose(src, dst):
            def dbody(d, _):
                rot = (d + lanes) & 15
                for e_blk in range(EMB // 16):
                    col = rot + e_blk * 16
                    vs = [plsc.load_gather(src, [rowvs[g], col])
                          for g in range(BCH // 16)]
                    for g in range(BCH // 16):
                        plsc.store_scatter(dst, [col, rowvs[g]], vs[g])
                return 0
            lax.fori_loop(0, 16, dbody, 0)

        def body(i, _):
            j0 = i * 4
            for t in range(4):
                j = j0 + t
                tb = t % 2
                pltpu.make_async_copy(
                    table_hbm.at[pl.ds(0, BCH)], rows[t], gsems[t]).wait()

                t3 = (t + 3) % 4
                @pl.when(j + 3 < seq)
                def _():
                    pltpu.async_copy(
                        table_hbm.at[idx_tv.at[j + 3]], rows[t3], gsems[t3])

                @pl.when(j >= 2)
                def _():
                    pltpu.make_async_copy(
                        trans[tb],
                        out_hbm.at[0, :, pl.ds(0, BCH)], wsems[tb]).wait()

                transpose(rows[t], trans[tb])
                pltpu.async_copy(
                    trans[tb], out_hbm.at[j, :, pl.ds(b0, BCH)], wsems[tb])
            return 0

        lax.fori_loop(0, seq // 4, body, 0)
        for tb in range(2):
            pltpu.make_async_copy(
                trans[tb], out_hbm.at[0, :, pl.ds(0, BCH)], wsems[tb]).wait()

    return gather_kernel


def kernel(input_tensor, token_emb_weight, ln_weight, ln_bias):
    batch, seq = input_tensor.shape
    table_t = jnp.transpose(token_emb_weight)
    normed = _normalize_table(table_t, ln_weight, ln_bias)
    idx_t = jnp.transpose(input_tensor)
    out_t = _make_gather(batch, seq)(normed, idx_t)
    return jnp.transpose(out_t, (2, 0, 1))
